# baseline (speedup 1.0000x reference)
import jax
import jax.numpy as jnp
from jax.experimental import pallas as pl

N_GRID_C = 324
N_MESH_C = 81


class _SubRef:
    def __init__(self, ref, off, rows, cols):
        self._ref = ref
        self._off = off
        self.shape = (rows, cols)

    def __getitem__(self, key):
        off, (rows, cols) = self._off, self.shape
        if key == slice(None):
            return self._ref[off : off + rows, 0:cols]
        return self._ref[off + key.start : off + key.stop, 0:cols]


class _RowRef:
    def __init__(self, ref, off, n):
        self._ref = ref
        self._off = off
        self.shape = (1, n)

    def __getitem__(self, key):
        return self._ref[0:1, self._off : self._off + self.shape[1]]


def _silu(x):
    return x * jax.lax.logistic(x)


def _ln(x, lnp):
    s, b = lnp
    mu = jnp.mean(x, axis=-1, keepdims=True)
    var = jnp.mean(jnp.square(x - mu), axis=-1, keepdims=True)
    return (x - mu) * jax.lax.rsqrt(var + 1e-5) * s[:] + b[:]


def _mm(a, b):
    return jnp.dot(a, b, preferred_element_type=jnp.float32)


def _mlp(p, x):
    layers = p["layers"]
    n = len(layers)
    for li, (Wr, br) in enumerate(layers):
        x = _mm(x, Wr[:]) + br[:]
        if li < n - 1:
            x = _silu(x)
    if "ln" in p:
        x = _ln(x, p["ln"])
    return x


def _tail(players, z, pln):
    for li in (1, 2):
        Wr, br = players[li]
        z = _mm(z, Wr[:]) + br[:]
        if li < 2:
            z = _silu(z)
    return _ln(z, pln)


def _mp_block(p, h_src, h_dst, e, gather_src, gather_dst, scatter, zero_dst):
    pe = p["edge"]["layers"]
    W0r, b0r = pe[0]
    z = gather_src(_mm(h_src, W0r[0:128]))
    z = z + _mm(e, W0r[256:384]) + b0r[:]
    if not zero_dst:
        z = z + gather_dst(_mm(h_dst, W0r[128:256]))
    z = _silu(z)
    e_new = e + _tail(pe, z, p["edge"]["ln"])

    agg = scatter(e_new)

    pn = p["node"]["layers"]
    V0r, c0r = pn[0]
    y = _mm(agg, V0r[128:256]) + c0r[:]
    if not zero_dst:
        y = y + _mm(h_dst, V0r[0:128])
    y = _silu(y)
    y = _tail(pn, y, p["node"]["ln"])
    h_new = y if zero_dst else h_dst + y
    return h_new, e_new


def _onehot(col_ref, n):
    e = col_ref.shape[0]
    ids = jax.lax.broadcasted_iota(jnp.int32, (e, n), 1)
    return (ids == col_ref[:]).astype(jnp.float32)


def _onehot_t(row_ref, n):
    e = row_ref.shape[1]
    ids = jax.lax.broadcasted_iota(jnp.int32, (n, e), 0)
    return (ids == row_ref[:]).astype(jnp.float32)


def _forecasters_lockstep(Ps, xs, attr_vals, oh):
    n = len(Ps)
    g2m, m2m, m2g = attr_vals
    h_g = [_mlp(Ps[i]["enc_node"], xs[i]) for i in range(n)]
    e = [_mlp(Ps[i]["enc_edge"], g2m) for i in range(n)]
    h_m = [None] * n
    for i in range(n):
        h_m[i], e[i] = _mp_block(
            Ps[i]["enc_blk"], h_g[i], None, e[i],
            lambda t: _mm(oh["g2m_src"], t), None,
            lambda t: _mm(oh["g2m_dst_t"], t),
            zero_dst=True,
        )
    em = [_mlp(Ps[i]["m2m_edge"], m2m) for i in range(n)]
    for bi in range(3):
        for i in range(n):
            h_m[i], em[i] = _mp_block(
                Ps[i]["proc"][bi], h_m[i], h_m[i], em[i],
                lambda t: _mm(oh["m2m_src"], t),
                lambda t: _mm(oh["m2m_dst"], t),
                lambda t: _mm(oh["m2m_dst_t"], t),
                zero_dst=False,
            )
    ed = [_mlp(Ps[i]["dec_edge"], m2g) for i in range(n)]
    for i in range(n):
        h_g[i], ed[i] = _mp_block(
            Ps[i]["dec_blk"], h_m[i], h_g[i], ed[i],
            lambda t: _mm(oh["m2g_src"], t),
            lambda t: _mm(oh["m2g_dst"], t),
            lambda t: _mm(oh["m2g_dst_t"], t),
            zero_dst=False,
        )
    return [xs[i] + _mlp(Ps[i]["dec_out"], h_g[i]) for i in range(n)]


def _pack_f32(leaves):
    parts, metas, off = [], [], 0
    for leaf in leaves:
        a = leaf if leaf.ndim == 2 else leaf[None, :]
        rows, cols = a.shape
        rp = -rows % 8
        parts.append(jnp.pad(a, ((0, rp), (0, 128 - cols))))
        metas.append((off, rows, cols))
        off += rows + rp
    return jnp.concatenate(parts, axis=0), metas


def kernel(features, params1, params2, params3, p1, p2, p3, g2m_attr, m2m_attr,
           m2g_attr, g2m_src, g2m_dst, m2m_src, m2m_dst, m2g_src, m2g_dst):
    ps = jnp.stack([p1, p2, p3]).astype(jnp.float32).reshape(3, 1)

    p_leaves, p_treedef = jax.tree.flatten((params1, params2, params3))
    big_p, p_metas = _pack_f32(p_leaves)
    attr_leaves = [g2m_attr, m2m_attr, m2g_attr]
    big_a, a_metas = _pack_f32(attr_leaves)

    col_arrs = [g2m_src, m2m_src, m2m_dst, m2g_src, m2g_dst]
    big_c, c_metas = _pack_f32([a[:, None] for a in col_arrs])
    row_arrs = [g2m_dst, m2m_dst, m2g_dst]
    row_offs, roff = [], 0
    for a in row_arrs:
        row_offs.append((roff, a.shape[0]))
        roff += a.shape[0]
    big_r = jnp.concatenate(row_arrs)[None, :]

    def _body(x_ref, ps_ref, pp_ref, aa_ref, cc_ref, rr_ref, out_ref):
        P1, P2, P3 = jax.tree.unflatten(
            p_treedef, [_SubRef(pp_ref, *m) for m in p_metas]
        )
        attr_vals = [_SubRef(aa_ref, *m)[:] for m in a_metas]
        cols = [_SubRef(cc_ref, *m) for m in c_metas]
        g2m_src_c, m2m_src_c, m2m_dst_c, m2g_src_c, m2g_dst_c = cols
        rows = [_RowRef(rr_ref, o, n) for (o, n) in row_offs]
        g2m_dst_r, m2m_dst_r, m2g_dst_r = rows
        oh = {
            "g2m_src": _onehot(g2m_src_c, N_GRID_C),
            "g2m_dst_t": _onehot_t(g2m_dst_r, N_MESH_C),
            "m2m_src": _onehot(m2m_src_c, N_MESH_C),
            "m2m_dst": _onehot(m2m_dst_c, N_MESH_C),
            "m2m_dst_t": _onehot_t(m2m_dst_r, N_MESH_C),
            "m2g_src": _onehot(m2g_src_c, N_MESH_C),
            "m2g_dst": _onehot(m2g_dst_c, N_GRID_C),
            "m2g_dst_t": _onehot_t(m2g_dst_r, N_GRID_C),
        }
        xs = [x_ref[0, mi] for mi in range(3)]
        outs = _forecasters_lockstep((P1, P2, P3), xs, attr_vals, oh)
        acc = None
        for mi in range(3):
            w = ps_ref[mi : mi + 1, :]
            acc = outs[mi] * w if acc is None else acc + outs[mi] * w
        out_ref[0] = acc

    out = pl.pallas_call(
        _body,
        out_shape=jax.ShapeDtypeStruct((1, N_GRID_C, 42), jnp.float32),
    )(features, ps, big_p, big_a, big_c, big_r)
    return out
